# baseline (speedup 1.0000x reference)
import math

import jax
import jax.numpy as jnp
from jax import lax
from jax.experimental import pallas as pl
from jax.experimental.pallas import tpu as pltpu
from jax.experimental.pallas import tpu_sc as plsc

N_SUM_NODES = 10000
N_CHILDREN = 50000
N_NNZ = 160000
BATCH_N = 128

SC_CORES = 2
SC_SUBCORES = 16
SC_LANES = 16
N_WORKERS = SC_CORES * SC_SUBCORES

NNZ_PAD = 163840
PER_WORKER = NNZ_PAD // N_WORKERS
CHUNK = 128
GDEPTH = 2
N_GROUPS = PER_WORKER // (CHUNK * GDEPTH)
N_SUM_PAD = 10240
ROWS_PER_TILE = N_SUM_PAD // SC_SUBCORES
ZERO_BLK = 128
Z_ROWS = N_SUM_PAD // BATCH_N

_PDF_BLK = 5000
_LW_BLK = NNZ_PAD // 128 // 10
_FIN_BLK = 1024
_EXP_BLK = 4096
_HALF_LOG_2PI = 0.5 * math.log(2.0 * math.pi)


def _prep_body(x_ref, locs_ref, scales_ref, lw_ref, p_ref, ew_ref):
    s = scales_ref[...] + 0.5
    z = (x_ref[...] - locs_ref[...]) / s
    ll = -0.5 * z * z - jnp.log(s) - _HALF_LOG_2PI
    p_ref[...] = jnp.exp(ll)
    ew_ref[...] = jnp.exp(lw_ref[...])


def _expb_body(w_ref, out_ref):
    out_ref[...] = jnp.broadcast_to(jnp.exp(w_ref[...]),
                                    (_EXP_BLK, SC_LANES))


def _fin_body(acc_ref, zacc_ref, out_ref):
    a = acc_ref[0] + acc_ref[1]
    zblk = (zacc_ref[0] + zacc_ref[1]).reshape(_FIN_BLK)
    out_ref[...] = jnp.log(a) - jnp.log(zblk)[:, None]


def _sc_accum_body(ew_hbm, ewb_hbm, rows_hbm, cols_hbm, p_hbm,
                   out_hbm, zout_hbm,
                   cols_v, rows_v, ew_v, wb_v, gath_v, zloc_v,
                   ziota_v, acc_sh, zsh, sem):
    c = lax.axis_index("c")
    s = lax.axis_index("s")
    wid = s * SC_CORES + c

    def _zrow(i, _):
        for q in range(BATCH_N // SC_LANES):
            gath_v[i, pl.ds(q * SC_LANES, SC_LANES)] = jnp.zeros(
                (SC_LANES,), jnp.float32)
        return 0

    lax.fori_loop(0, ZERO_BLK, _zrow, 0)
    for k in range(ROWS_PER_TILE // ZERO_BLK):
        pltpu.sync_copy(
            gath_v.at[pl.ds(0, ZERO_BLK)],
            acc_sh.at[pl.ds(s * ROWS_PER_TILE + k * ZERO_BLK, ZERO_BLK)])

    def _zlrow(i, _):
        for q in range(BATCH_N // SC_LANES):
            zloc_v[i, pl.ds(q * SC_LANES, SC_LANES)] = jnp.zeros(
                (SC_LANES,), jnp.float32)
        return 0

    lax.fori_loop(0, Z_ROWS, _zlrow, 0)
    for k in range(Z_ROWS // SC_LANES):
        ziota_v[pl.ds(k * SC_LANES, SC_LANES)] = (
            lax.iota(jnp.int32, SC_LANES) + (k * SC_LANES))

    @pl.when(s == 0)
    def _():
        pltpu.sync_copy(gath_v.at[pl.ds(0, Z_ROWS)], zsh)

    plsc.subcore_barrier()

    def _group(g, _):
        gbase = wid * PER_WORKER + g * (GDEPTH * CHUNK)
        descs = []
        for b in range(GDEPTH):
            base = gbase + b * CHUNK
            pltpu.sync_copy(cols_hbm.at[pl.ds(base, CHUNK)], cols_v.at[b])
            pltpu.sync_copy(rows_hbm.at[pl.ds(base, CHUNK)], rows_v.at[b])
            pltpu.sync_copy(ew_hbm.at[pl.ds(base, CHUNK)], ew_v.at[b])
            pltpu.sync_copy(
                ewb_hbm.at[pl.ds(base * SC_LANES, CHUNK * SC_LANES)],
                wb_v.at[b])
            descs.append(pltpu.async_copy(
                p_hbm.at[cols_v.at[b]],
                gath_v.at[pl.ds(b * CHUNK, CHUNK)], sem))
        for d in descs:
            d.wait()
        for b in range(GDEPTH):
            def _mulrow(j, _, b=b):
                off = pl.multiple_of(j * SC_LANES, SC_LANES)
                wsp = wb_v[b, pl.ds(off, SC_LANES)]
                for q in range(BATCH_N // SC_LANES):
                    sl = pl.ds(q * SC_LANES, SC_LANES)
                    gath_v[b * CHUNK + j, sl] = gath_v[b * CHUNK + j, sl] * wsp
                return 0

            lax.fori_loop(0, CHUNK, _mulrow, 0)
            pltpu.sync_copy(gath_v.at[pl.ds(b * CHUNK, CHUNK)],
                            acc_sh.at[rows_v.at[b]], add=True)
            for q in range(CHUNK // SC_LANES):
                sl = pl.ds(q * SC_LANES, SC_LANES)
                idx = rows_v[b, sl]
                plsc.addupdate_scatter(
                    zloc_v,
                    [lax.shift_right_logical(idx, 7),
                     lax.bitwise_and(idx, jnp.int32(127))],
                    ew_v[b, sl])
        return 0

    lax.fori_loop(0, N_GROUPS, _group, 0)

    pltpu.sync_copy(zloc_v, zsh.at[ziota_v], add=True)
    plsc.subcore_barrier()

    pltpu.sync_copy(
        acc_sh.at[pl.ds(s * ROWS_PER_TILE, ROWS_PER_TILE)],
        out_hbm.at[c, pl.ds(s * ROWS_PER_TILE, ROWS_PER_TILE)])

    @pl.when(s == 0)
    def _():
        pltpu.sync_copy(zsh, zout_hbm.at[c])


def _make_sc_accum():
    return pl.kernel(
        _sc_accum_body,
        out_type=(
            jax.ShapeDtypeStruct((SC_CORES, N_SUM_PAD, BATCH_N), jnp.float32),
            jax.ShapeDtypeStruct((SC_CORES, Z_ROWS, BATCH_N), jnp.float32),
        ),
        mesh=plsc.VectorSubcoreMesh(core_axis_name="c", subcore_axis_name="s",
                                    num_cores=SC_CORES,
                                    num_subcores=SC_SUBCORES),
        compiler_params=pltpu.CompilerParams(needs_layout_passes=False),
        scratch_types=[
            pltpu.VMEM((GDEPTH, CHUNK), jnp.int32),
            pltpu.VMEM((GDEPTH, CHUNK), jnp.int32),
            pltpu.VMEM((GDEPTH, CHUNK), jnp.float32),
            pltpu.VMEM((GDEPTH, CHUNK * SC_LANES), jnp.float32),
            pltpu.VMEM((GDEPTH * CHUNK, BATCH_N), jnp.float32),
            pltpu.VMEM((Z_ROWS, BATCH_N), jnp.float32),
            pltpu.VMEM((Z_ROWS,), jnp.int32),
            pltpu.VMEM_SHARED((N_SUM_PAD, BATCH_N), jnp.float32),
            pltpu.VMEM_SHARED((Z_ROWS, BATCH_N), jnp.float32),
            pltpu.SemaphoreType.DMA,
        ],
    )


def kernel(x, locs, scales, log_weight_data, rows, cols):
    pad = NNZ_PAD - N_NNZ
    lw_p = jnp.concatenate(
        [log_weight_data, jnp.full(pad, -1e30, jnp.float32)])
    p_table, ew2d = pl.pallas_call(
        _prep_body,
        grid=(N_CHILDREN // _PDF_BLK,),
        in_specs=[
            pl.BlockSpec((1, BATCH_N), lambda i: (0, 0)),
            pl.BlockSpec((_PDF_BLK, 1), lambda i: (i, 0)),
            pl.BlockSpec((_PDF_BLK, 1), lambda i: (i, 0)),
            pl.BlockSpec((_LW_BLK, 128), lambda i: (i, 0)),
        ],
        out_specs=[
            pl.BlockSpec((_PDF_BLK, BATCH_N), lambda i: (i, 0)),
            pl.BlockSpec((_LW_BLK, 128), lambda i: (i, 0)),
        ],
        out_shape=[
            jax.ShapeDtypeStruct((N_CHILDREN, BATCH_N), jnp.float32),
            jax.ShapeDtypeStruct((NNZ_PAD // 128, 128), jnp.float32),
        ],
    )(x.reshape(1, BATCH_N), locs.reshape(N_CHILDREN, 1),
      scales.reshape(N_CHILDREN, 1), lw_p.reshape(NNZ_PAD // 128, 128))

    ewb = pl.pallas_call(
        _expb_body,
        grid=(NNZ_PAD // _EXP_BLK,),
        in_specs=[pl.BlockSpec((_EXP_BLK, 1), lambda i: (i, 0))],
        out_specs=pl.BlockSpec((_EXP_BLK, SC_LANES), lambda i: (i, 0)),
        out_shape=jax.ShapeDtypeStruct((NNZ_PAD, SC_LANES), jnp.float32),
    )(lw_p.reshape(NNZ_PAD, 1))

    rows_p = jnp.concatenate([rows, jnp.zeros(pad, jnp.int32)])
    cols_p = jnp.concatenate([cols, jnp.zeros(pad, jnp.int32)])

    acc, zacc = _make_sc_accum()(ew2d.reshape(NNZ_PAD),
                                 ewb.reshape(NNZ_PAD * SC_LANES),
                                 rows_p, cols_p, p_table)

    out_t = pl.pallas_call(
        _fin_body,
        grid=(N_SUM_PAD // _FIN_BLK,),
        in_specs=[
            pl.BlockSpec((SC_CORES, _FIN_BLK, BATCH_N), lambda j: (0, j, 0)),
            pl.BlockSpec((SC_CORES, _FIN_BLK // BATCH_N, BATCH_N),
                         lambda j: (0, j, 0)),
        ],
        out_specs=pl.BlockSpec((_FIN_BLK, BATCH_N), lambda j: (j, 0)),
        out_shape=jax.ShapeDtypeStruct((N_SUM_PAD, BATCH_N), jnp.float32),
    )(acc, zacc)
    return out_t[:N_SUM_NODES].T
